# baseline (speedup 1.0000x reference)
import functools
from math import sqrt

import jax
import jax.numpy as jnp
import numpy as np
from jax.experimental import pallas as pl
from jax.experimental.pallas import tpu as pltpu

FACTOR = 5
QB = 512
HP = 2

_PREC = jax.lax.Precision.HIGHEST


def _phase1_body(H, D, idx_ref, q_ref, k_ref, m_ref, cnt_ref, neg_ref):
    qb, L_K = cnt_ref.shape
    S = idx_ref.shape[1]

    jota = jax.lax.broadcasted_iota(jnp.int32, (qb, L_K), 1)
    cnt = jnp.zeros((qb, L_K), jnp.float32)
    for s in range(S):
        col = idx_ref[:, s].reshape(qb, 1)
        cnt = cnt + (jota == col).astype(jnp.float32)
    cnt_ref[...] = cnt
    neg_ref[...] = jnp.where(cnt > 0.0, 0.0, -1e30)

    for h in range(H):
        q = q_ref[:, h * D:(h + 1) * D]
        k = k_ref[:, h * D:(h + 1) * D]
        scores = jax.lax.dot_general(
            q, k, (((1,), (1,)), ((), ())),
            preferred_element_type=jnp.float32, precision=_PREC)
        ck = jax.lax.dot_general(
            cnt_ref[...], k, (((1,), (0,)), ((), ())),
            preferred_element_type=jnp.float32, precision=_PREC)
        maxt = jnp.max(scores + neg_ref[...], axis=1)
        sumt = jnp.sum(q * ck, axis=1)
        m_ref[h, :] = maxt - sumt / L_K


def _phase2_body(u, scale, D, m_ref, q_ref, k_ref, v_ref, out_ref, oh_ref):
    L = m_ref.shape[2]
    hp = m_ref.shape[0]

    m = m_ref[:, 0, :]
    lane = jax.lax.broadcasted_iota(jnp.int32, (hp, L), 1)
    for uu in range(u):
        cur = jnp.max(m, axis=1, keepdims=True)
        am = jnp.min(jnp.where(m == cur, lane, L), axis=1, keepdims=True)
        sel = lane == am
        for p in range(hp):
            oh_ref[p, uu, :] = sel[p, :].astype(jnp.float32)
        m = jnp.where(sel, -1e30, m)

    for p in range(hp):
        oh = oh_ref[p]
        q = q_ref[:, p * D:(p + 1) * D]
        k = k_ref[:, p * D:(p + 1) * D]
        v = v_ref[:, p * D:(p + 1) * D]
        q_sel = jax.lax.dot_general(
            oh, q, (((1,), (0,)), ((), ())),
            preferred_element_type=jnp.float32, precision=_PREC)
        scores = jax.lax.dot_general(
            q_sel, k, (((1,), (1,)), ((), ())),
            preferred_element_type=jnp.float32, precision=_PREC) * scale
        smax = jnp.max(scores, axis=1, keepdims=True)
        e = jnp.exp(scores - smax)
        attn = e / jnp.sum(e, axis=1, keepdims=True)
        upd = jax.lax.dot_general(
            attn, v, (((1,), (0,)), ((), ())),
            preferred_element_type=jnp.float32, precision=_PREC)
        scat = jax.lax.dot_general(
            oh, upd, (((0,), (0,)), ((), ())),
            preferred_element_type=jnp.float32, precision=_PREC)
        rowmask = jnp.sum(oh, axis=0).reshape(L, 1)
        meanv = jnp.sum(v, axis=0, keepdims=True) / L
        out_ref[:, p * D:(p + 1) * D] = scat + (1.0 - rowmask) * meanv


def kernel(queries, keys, values, attn_mask, index_sample):
    B, L, H, D = queries.shape
    L_K = keys.shape[1]
    S = index_sample.shape[1]
    u = min(FACTOR * int(np.ceil(np.log(L))), L)
    scale = 1.0 / sqrt(D)

    q2 = queries.reshape(L, H * D)
    k2 = keys.reshape(L_K, H * D)
    v2 = values.reshape(L_K, H * D)
    idx = index_sample.astype(jnp.int32)

    nqb = L // QB
    m = pl.pallas_call(
        functools.partial(_phase1_body, H, D),
        grid=(nqb,),
        in_specs=[
            pl.BlockSpec((QB, S), lambda i: (i, 0)),
            pl.BlockSpec((QB, H * D), lambda i: (i, 0)),
            pl.BlockSpec((L_K, H * D), lambda i: (0, 0)),
        ],
        out_specs=pl.BlockSpec((H, QB), lambda i: (0, i)),
        out_shape=jax.ShapeDtypeStruct((H, L), jnp.float32),
        scratch_shapes=[
            pltpu.VMEM((QB, L_K), jnp.float32),
            pltpu.VMEM((QB, L_K), jnp.float32),
        ],
    )(idx, q2, k2)

    out = pl.pallas_call(
        functools.partial(_phase2_body, u, scale, D),
        grid=(H // HP,),
        in_specs=[
            pl.BlockSpec((HP, 1, L), lambda h: (h, 0, 0)),
            pl.BlockSpec((L, HP * D), lambda h: (0, h)),
            pl.BlockSpec((L_K, HP * D), lambda h: (0, h)),
            pl.BlockSpec((L_K, HP * D), lambda h: (0, h)),
        ],
        out_specs=pl.BlockSpec((L, HP * D), lambda h: (0, h)),
        out_shape=jax.ShapeDtypeStruct((L, H * D), jnp.float32),
        scratch_shapes=[
            pltpu.VMEM((HP, u, L_K), jnp.float32),
        ],
    )(m.reshape(H, 1, L), q2, k2, v2)

    return out.reshape(B, L, H, D)
